# Initial kernel scaffold; baseline (speedup 1.0000x reference)
import jax
import jax.numpy as jnp
from jax.experimental import pallas as pl


def kernel(x, edge_index, batch, sheet_node_idx, sheet_features, W0, b0, W1, b1, W2, b2, Wg1, bg1, Wg2, bg2, Wf, bf, Wq1, bq1, Wq2, bq2):
    raise NotImplementedError("write your pallas kernel here")


# speedup vs baseline: 16.2428x; 16.2428x over previous
import jax
import jax.numpy as jnp
from jax import lax
from jax.experimental import pallas as pl
from jax.experimental.pallas import tpu as pltpu
from jax.experimental.pallas import tpu_sc as plsc

NN = 100000
EE = 1600000
HID = 64
NSHEET = 256
LSHEET = 128

ACC_ROWS = 102400
ACC_DEG = 131072
PAD_ROW = 100000
EROWS = 12544
WIN = 49
NB = 6
DP = 4


def _ring(J, table, acc, sbuf, dbuf, rbuf, gsem, ssem):
    gd = [None] * J
    sd = [None] * J

    def gather(k):
        return pltpu.async_copy(table.at[sbuf.at[k]], rbuf.at[k % NB], gsem)

    def scatter(j):
        return pltpu.async_copy(rbuf.at[j % NB], acc.at[dbuf.at[j]], ssem,
                                add=True)

    for k in range(min(DP, J)):
        gd[k] = gather(k)
    for j in range(J):
        gd[j].wait()
        sd[j] = scatter(j)
        k = j + DP
        if k < J:
            if k - NB >= 0:
                sd[k - NB].wait()
            gd[k] = gather(k)
    for j in range(max(0, J - NB), J):
        sd[j].wait()


def _deg_body(dstp, out0, out1, idxb, ones, zbuf, acc, ssem):
    c = lax.axis_index("c")
    s = lax.axis_index("s")

    def zb(i, carry):
        zbuf[pl.ds(i * 16, 16)] = jnp.zeros((16,), jnp.float32)
        return carry
    lax.fori_loop(0, 512, zb, None)
    for i in range(8):
        ones[pl.ds(i * 16, 16)] = jnp.ones((16,), jnp.float32)
    pltpu.sync_copy(zbuf, acc.at[pl.ds(s * 8192, 8192)])
    plsc.subcore_barrier()

    base = c * 6272 + s * 392

    def win(w, carry):
        pltpu.sync_copy(dstp.at[pl.ds(base + w * 8, 8)], idxb)
        sds = [pltpu.async_copy(ones, acc.at[idxb.at[j]], ssem, add=True)
               for j in range(8)]
        for d in sds:
            d.wait()
        return carry
    lax.fori_loop(0, WIN, win, None)
    plsc.subcore_barrier()

    pltpu.sync_copy(acc.at[pl.ds(s * 8192, 8192)], zbuf)

    @pl.when(c == 0)
    def _():
        pltpu.sync_copy(zbuf, out0.at[pl.ds(s * 8192, 8192)])

    @pl.when(c == 1)
    def _():
        pltpu.sync_copy(zbuf, out1.at[pl.ds(s * 8192, 8192)])


def _prop1_body(vp1, srcp, dstp, out, sbuf, dbuf, rbuf, vstage, acc,
                gsem, ssem):
    c = lax.axis_index("c")
    s = lax.axis_index("s")

    def zrow(i, carry):
        vstage[i, :] = jnp.zeros((16,), jnp.float32)
        return carry
    lax.fori_loop(0, 640, zrow, None)
    for p in range(10):
        pltpu.sync_copy(vstage, acc.at[pl.ds(s * 6400 + p * 640, 640)])
    plsc.subcore_barrier()

    base = c * 6272 + s * 392

    def win(w, carry):
        r0 = base + w * 8
        pltpu.sync_copy(srcp.at[pl.ds(r0, 8)], sbuf)
        pltpu.sync_copy(dstp.at[pl.ds(r0, 8)], dbuf)
        _ring(8, vp1, acc, sbuf, dbuf, rbuf, gsem, ssem)
        return carry
    lax.fori_loop(0, WIN, win, None)
    plsc.subcore_barrier()

    for p in range(10):
        r0 = s * 6400 + p * 640
        pltpu.sync_copy(acc.at[pl.ds(r0, 640)], vstage)
        pltpu.sync_copy(vstage, out.at[c].at[pl.ds(r0, 640)])


def _prop23_body(vps, srcp, dstp, out, sbuf, dbuf, rbuf, vstage, acc,
                 gsem, ssem):
    c = lax.axis_index("c")
    s = lax.axis_index("s")

    for i in range(2):
        cc = c * 2 + i
        for p in range(10):
            r0 = s * 6400 + p * 640
            pltpu.sync_copy(vps.at[cc].at[pl.ds(r0, 640)], vstage)
            pltpu.sync_copy(vstage, acc.at[pl.ds(r0, 640)])
        plsc.subcore_barrier()

        base = s * 784

        def win(w, carry):
            r0 = base + w * 16
            pltpu.sync_copy(srcp.at[pl.ds(r0, 16)], sbuf)
            pltpu.sync_copy(dstp.at[pl.ds(r0, 16)], dbuf)
            _ring(16, vps.at[cc], acc, sbuf, dbuf, rbuf, gsem, ssem)
            return carry
        lax.fori_loop(0, WIN, win, None)
        plsc.subcore_barrier()

        for p in range(10):
            r0 = s * 6400 + p * 640
            pltpu.sync_copy(acc.at[pl.ds(r0, 640)], vstage)
            pltpu.sync_copy(vstage, out.at[cc].at[pl.ds(r0, 640)])
        if i == 0:
            plsc.subcore_barrier()


def _sheet_body(h3, sidx, sdstl, out, sbuf, dbuf, rbuf, zstage, acc,
                gsem, ssem):
    c = lax.axis_index("c")
    s = lax.axis_index("s")

    for i in range(8):
        for q in range(4):
            zstage[i, pl.ds(q * 16, 16)] = jnp.zeros((16,), jnp.float32)
    pltpu.sync_copy(zstage, acc.at[pl.ds(s * 8, 8)])
    plsc.subcore_barrier()

    r0 = c * 128 + s * 8
    pltpu.sync_copy(sidx.at[pl.ds(r0, 8)], sbuf)
    pltpu.sync_copy(sdstl.at[pl.ds(r0, 8)], dbuf)
    _ring(8, h3, acc, sbuf, dbuf, rbuf, gsem, ssem)
    plsc.subcore_barrier()

    pltpu.sync_copy(acc.at[pl.ds(s * 8, 8)], zstage)
    pltpu.sync_copy(zstage, out.at[pl.ds(r0, 8)])


_MESH = None


def _mesh():
    global _MESH
    if _MESH is None:
        _MESH = plsc.VectorSubcoreMesh(core_axis_name="c",
                                       subcore_axis_name="s", num_cores=2,
                                       num_subcores=16)
    return _MESH


def _sc_deg(dstp):
    return pl.kernel(
        _deg_body,
        out_type=[jax.ShapeDtypeStruct((ACC_DEG,), jnp.float32),
                  jax.ShapeDtypeStruct((ACC_DEG,), jnp.float32)],
        mesh=_mesh(),
        compiler_params=pltpu.CompilerParams(use_tc_tiling_on_sc=False),
        scratch_types=[
            pltpu.VMEM((8, 128), jnp.int32),
            pltpu.VMEM((128,), jnp.float32),
            pltpu.VMEM((8192,), jnp.float32),
            pltpu.VMEM_SHARED((ACC_DEG,), jnp.float32),
            pltpu.SemaphoreType.DMA,
        ],
    )(dstp)


def _sc_prop1(vp1, srcp, dstp):
    return pl.kernel(
        _prop1_body,
        out_type=jax.ShapeDtypeStruct((2, ACC_ROWS, 16), jnp.float32),
        mesh=_mesh(),
        compiler_params=pltpu.CompilerParams(use_tc_tiling_on_sc=False),
        scratch_types=[
            pltpu.VMEM((8, 128), jnp.int32),
            pltpu.VMEM((8, 128), jnp.int32),
            pltpu.VMEM((NB, 128, 16), jnp.float32),
            pltpu.VMEM((640, 16), jnp.float32),
            pltpu.VMEM_SHARED((ACC_ROWS, 16), jnp.float32),
            pltpu.SemaphoreType.DMA,
            pltpu.SemaphoreType.DMA,
        ],
    )(vp1, srcp, dstp)


def _sc_prop23(vps, srcp, dstp):
    return pl.kernel(
        _prop23_body,
        out_type=jax.ShapeDtypeStruct((4, ACC_ROWS, 16), jnp.float32),
        mesh=_mesh(),
        compiler_params=pltpu.CompilerParams(use_tc_tiling_on_sc=False),
        scratch_types=[
            pltpu.VMEM((16, 128), jnp.int32),
            pltpu.VMEM((16, 128), jnp.int32),
            pltpu.VMEM((NB, 128, 16), jnp.float32),
            pltpu.VMEM((640, 16), jnp.float32),
            pltpu.VMEM_SHARED((ACC_ROWS, 16), jnp.float32),
            pltpu.SemaphoreType.DMA,
            pltpu.SemaphoreType.DMA,
        ],
    )(vps, srcp, dstp)


def _sc_sheet(h3, sidx, sdstl):
    return pl.kernel(
        _sheet_body,
        out_type=jax.ShapeDtypeStruct((NSHEET, HID), jnp.float32),
        mesh=_mesh(),
        compiler_params=pltpu.CompilerParams(use_tc_tiling_on_sc=False),
        scratch_types=[
            pltpu.VMEM((8, 128), jnp.int32),
            pltpu.VMEM((8, 128), jnp.int32),
            pltpu.VMEM((NB, 128, HID), jnp.float32),
            pltpu.VMEM((8, HID), jnp.float32),
            pltpu.VMEM_SHARED((128, HID), jnp.float32),
            pltpu.SemaphoreType.DMA,
            pltpu.SemaphoreType.DMA,
        ],
    )(h3, sidx, sdstl)


_BLK = 1000
_GRID = NN // _BLK


def _prep_body(d0, d1, xp, dis, vp1):
    dv = lax.rsqrt(d0[...] + d1[...] + 1.0)
    dis[...] = dv
    vp1[...] = dv * xp[...]


def _tc_prep(d0, d1, xp):
    return pl.pallas_call(
        _prep_body,
        grid=(_GRID,),
        in_specs=[
            pl.BlockSpec((_BLK, 1), lambda i: (i, 0)),
            pl.BlockSpec((_BLK, 1), lambda i: (i, 0)),
            pl.BlockSpec((_BLK, 16), lambda i: (i, 0)),
        ],
        out_specs=[
            pl.BlockSpec((_BLK, 1), lambda i: (i, 0)),
            pl.BlockSpec((_BLK, 16), lambda i: (i, 0)),
        ],
        out_shape=[
            jax.ShapeDtypeStruct((NN, 1), jnp.float32),
            jax.ShapeDtypeStruct((NN, 16), jnp.float32),
        ],
    )(d0, d1, xp)


def _comb1_body(o, vp1, dis, w0, w1, b0, vp2):
    t = dis[...] * (o[0] + o[1] + vp1[...])
    h1 = jnp.maximum(jnp.dot(t, w0[...]) + b0[...], 0.0)
    v = dis[...] * jnp.dot(h1, w1[...])
    for cidx in range(4):
        vp2[cidx] = v[:, cidx * 16:(cidx + 1) * 16]


def _tc_comb1(o, vp1, dis, w0p, w1, b0):
    return pl.pallas_call(
        _comb1_body,
        grid=(_GRID,),
        in_specs=[
            pl.BlockSpec((2, _BLK, 16), lambda i: (0, i, 0)),
            pl.BlockSpec((_BLK, 16), lambda i: (i, 0)),
            pl.BlockSpec((_BLK, 1), lambda i: (i, 0)),
            pl.BlockSpec((16, HID), lambda i: (0, 0)),
            pl.BlockSpec((HID, HID), lambda i: (0, 0)),
            pl.BlockSpec((1, HID), lambda i: (0, 0)),
        ],
        out_specs=pl.BlockSpec((4, _BLK, 16), lambda i: (0, i, 0)),
        out_shape=jax.ShapeDtypeStruct((4, ACC_ROWS, 16), jnp.float32),
    )(o, vp1, dis, w0p, w1, b0)


def _comb2_body(o, dis, w2, b1, vp3):
    x = jnp.concatenate([o[i] for i in range(4)], axis=1)
    h2 = jnp.maximum(dis[...] * x + b1[...], 0.0)
    v = dis[...] * jnp.dot(h2, w2[...])
    for cidx in range(4):
        vp3[cidx] = v[:, cidx * 16:(cidx + 1) * 16]


def _tc_comb2(o, dis, w2, b1):
    return pl.pallas_call(
        _comb2_body,
        grid=(_GRID,),
        in_specs=[
            pl.BlockSpec((4, _BLK, 16), lambda i: (0, i, 0)),
            pl.BlockSpec((_BLK, 1), lambda i: (i, 0)),
            pl.BlockSpec((HID, HID), lambda i: (0, 0)),
            pl.BlockSpec((1, HID), lambda i: (0, 0)),
        ],
        out_specs=pl.BlockSpec((4, _BLK, 16), lambda i: (0, i, 0)),
        out_shape=jax.ShapeDtypeStruct((4, ACC_ROWS, 16), jnp.float32),
    )(o, dis, w2, b1)


def _comb3_body(o, dis, b2, h3, gs):
    x = jnp.concatenate([o[i] for i in range(4)], axis=1)
    h = jnp.maximum(dis[...] * x + b2[...], 0.0)
    h3[...] = h

    @pl.when(pl.program_id(0) == 0)
    def _():
        gs[...] = jnp.zeros_like(gs)
    gs[...] += jnp.sum(h, axis=0, keepdims=True)


def _tc_comb3(o, dis, b2):
    return pl.pallas_call(
        _comb3_body,
        grid=(_GRID,),
        in_specs=[
            pl.BlockSpec((4, _BLK, 16), lambda i: (0, i, 0)),
            pl.BlockSpec((_BLK, 1), lambda i: (i, 0)),
            pl.BlockSpec((1, HID), lambda i: (0, 0)),
        ],
        out_specs=[
            pl.BlockSpec((_BLK, HID), lambda i: (i, 0)),
            pl.BlockSpec((1, HID), lambda i: (0, 0)),
        ],
        out_shape=[
            jax.ShapeDtypeStruct((NN, HID), jnp.float32),
            jax.ShapeDtypeStruct((1, HID), jnp.float32),
        ],
    )(o, dis, b2)


def _mlp_body(ssum, gs, sfp, wg1, bg1, wg2, bg2, wfa, wfb, bf,
              wq1a, wq1b, bq1, wq2, bq2, out):
    se = ssum[...] * (1.0 / LSHEET)
    ge = gs[...] * (1.0 / NN)
    geo = jnp.dot(jnp.maximum(jnp.dot(sfp[...], wg1[...]) + bg1[...], 0.0),
                  wg2[...]) + bg2[...]
    fused = jnp.maximum(jnp.dot(se, wfa[...]) + jnp.dot(geo, wfb[...])
                        + bf[...], 0.0)
    gq = jnp.dot(ge, wq1b[...])
    z = jnp.maximum(jnp.dot(fused, wq1a[...]) + gq + bq1[...], 0.0)
    out[...] = jnp.dot(z, wq2[...]) + bq2[...]


def _tc_mlp(*args):
    return pl.pallas_call(
        _mlp_body,
        out_shape=jax.ShapeDtypeStruct((NSHEET, 1), jnp.float32),
    )(*args)


def kernel(x, edge_index, batch, sheet_node_idx, sheet_features,
           W0, b0, W1, b1, W2, b2, Wg1, bg1, Wg2, bg2, Wf, bf,
           Wq1, bq1, Wq2, bq2):
    i32 = jnp.int32
    src = edge_index[0]
    dst = edge_index[1]
    npad = EROWS * 128 - EE
    srcp = jnp.concatenate([src, jnp.zeros((npad,), i32)]).reshape(EROWS, 128)
    dstp = jnp.concatenate([dst, jnp.full((npad,), PAD_ROW, i32)]
                           ).reshape(EROWS, 128)
    xp = jnp.pad(x, ((0, 0), (0, 6)))
    w0p = jnp.pad(W0, ((0, 6), (0, 0)))
    wg1p = jnp.pad(Wg1, ((0, 6), (0, 0)))
    sfp = jnp.pad(sheet_features, ((0, 0), (0, 6)))
    sdstl = jnp.broadcast_to(
        (jnp.arange(NSHEET, dtype=i32) % 128)[:, None], (NSHEET, LSHEET))
    b0r = b0.reshape(1, HID)
    b1r = b1.reshape(1, HID)
    b2r = b2.reshape(1, HID)
    bg1r = bg1.reshape(1, HID)
    bg2r = bg2.reshape(1, HID)
    bfr = bf.reshape(1, HID)
    bq1r = bq1.reshape(1, HID)
    bq2r = bq2.reshape(1, 1)
    wfa, wfb = Wf[:HID], Wf[HID:]
    wq1a, wq1b = Wq1[:HID], Wq1[HID:]

    dg0, dg1 = _sc_deg(dstp)
    d0 = dg0[:NN].reshape(NN, 1)
    d1 = dg1[:NN].reshape(NN, 1)
    dis, vp1 = _tc_prep(d0, d1, xp)

    o1 = _sc_prop1(vp1, srcp, dstp)
    vp2 = _tc_comb1(o1, vp1, dis, w0p, W1, b0r)

    o2 = _sc_prop23(vp2, srcp, dstp)
    vp3 = _tc_comb2(o2, dis, W2, b1r)

    o3 = _sc_prop23(vp3, srcp, dstp)
    h3, gs = _tc_comb3(o3, dis, b2r)

    ssum = _sc_sheet(h3, sheet_node_idx, sdstl)
    q = _tc_mlp(ssum, gs, sfp, wg1p, bg1r, Wg2, bg2r, wfa, wfb, bfr,
                wq1a, wq1b, bq1r, Wq2, bq2r)
    return jnp.squeeze(q, -1)


# speedup vs baseline: 17.7346x; 1.0918x over previous
import jax
import jax.numpy as jnp
from jax import lax
from jax.experimental import pallas as pl
from jax.experimental.pallas import tpu as pltpu
from jax.experimental.pallas import tpu_sc as plsc

NN = 100000
EE = 1600000
HID = 64
NSHEET = 256
LSHEET = 128

ACC_ROWS = 100352
ACC_DEG = 131072
PAD_ROW = 100000
EROWS = 3136
WIN = 49
NB = 3
DP = 2


def _ring(J, table, acc, sbuf, dbuf, rbuf, gsem, ssem):
    gd = [None] * J
    sd = [None] * J

    def gather(k):
        return pltpu.async_copy(table.at[sbuf.at[k]], rbuf.at[k % NB], gsem)

    def scatter(j):
        return pltpu.async_copy(rbuf.at[j % NB], acc.at[dbuf.at[j]], ssem,
                                add=True)

    for k in range(min(DP, J)):
        gd[k] = gather(k)
    for j in range(J):
        gd[j].wait()
        sd[j] = scatter(j)
        k = j + DP
        if k < J:
            if k - NB >= 0:
                sd[k - NB].wait()
            gd[k] = gather(k)
    for j in range(max(0, J - NB), J):
        sd[j].wait()


def _deg_body(dstp, out0, out1, idxb, ones, zbuf, acc, ssem):
    c = lax.axis_index("c")
    s = lax.axis_index("s")

    def zb(i, carry):
        zbuf[pl.ds(i * 16, 16)] = jnp.zeros((16,), jnp.float32)
        return carry
    lax.fori_loop(0, 512, zb, None)
    def ob(i, carry):
        ones[pl.ds(i * 16, 16)] = jnp.ones((16,), jnp.float32)
        return carry
    lax.fori_loop(0, 32, ob, None)
    pltpu.sync_copy(zbuf, acc.at[pl.ds(s * 8192, 8192)])
    plsc.subcore_barrier()

    base = c * 1568 + s * 98

    def win(w, carry):
        pltpu.sync_copy(dstp.at[pl.ds(base + w * 2, 2)], idxb)
        d0 = pltpu.async_copy(ones, acc.at[idxb.at[0]], ssem, add=True)
        d1 = pltpu.async_copy(ones, acc.at[idxb.at[1]], ssem, add=True)
        d0.wait()
        d1.wait()
        return carry
    lax.fori_loop(0, WIN, win, None)
    plsc.subcore_barrier()

    pltpu.sync_copy(acc.at[pl.ds(s * 8192, 8192)], zbuf)

    @pl.when(c == 0)
    def _():
        pltpu.sync_copy(zbuf, out0.at[pl.ds(s * 8192, 8192)])

    @pl.when(c == 1)
    def _():
        pltpu.sync_copy(zbuf, out1.at[pl.ds(s * 8192, 8192)])


def _prop1_body(vp1, srcp, dstp, out, sbuf, dbuf, rbuf, acc, gsem, ssem):
    c = lax.axis_index("c")
    s = lax.axis_index("s")

    rows = pl.ds(s * 6272, 6272)
    pltpu.sync_copy(vp1.at[rows], acc.at[rows])
    plsc.subcore_barrier()

    base = c * 1568 + s * 98

    def win(w, carry):
        r0 = base + w * 2
        pltpu.sync_copy(srcp.at[pl.ds(r0, 2)], sbuf)
        pltpu.sync_copy(dstp.at[pl.ds(r0, 2)], dbuf)
        _ring(2, vp1, acc, sbuf, dbuf, rbuf, gsem, ssem)
        return carry
    lax.fori_loop(0, WIN, win, None)
    plsc.subcore_barrier()

    pltpu.sync_copy(acc.at[rows], out.at[c].at[rows])


def _prop23_body(vps, srcp, dstp, out, sbuf, dbuf, rbuf, acc, gsem, ssem):
    c = lax.axis_index("c")
    s = lax.axis_index("s")

    rows = pl.ds(s * 6272, 6272)
    for i in range(2):
        cc = c * 2 + i
        pltpu.sync_copy(vps.at[cc].at[rows], acc.at[rows])
        plsc.subcore_barrier()

        base = s * 196

        def win(w, carry):
            r0 = base + w * 4
            pltpu.sync_copy(srcp.at[pl.ds(r0, 4)], sbuf)
            pltpu.sync_copy(dstp.at[pl.ds(r0, 4)], dbuf)
            _ring(4, vps.at[cc], acc, sbuf, dbuf, rbuf, gsem, ssem)
            return carry
        lax.fori_loop(0, WIN, win, None)
        plsc.subcore_barrier()

        pltpu.sync_copy(acc.at[rows], out.at[cc].at[rows])
        if i == 0:
            plsc.subcore_barrier()


def _sheet_body(h3, sidx, sdstl, out, sbuf, dbuf, rbuf, zstage, acc,
                gsem, ssem):
    c = lax.axis_index("c")
    s = lax.axis_index("s")

    for i in range(8):
        for q in range(4):
            zstage[i, pl.ds(q * 16, 16)] = jnp.zeros((16,), jnp.float32)
    pltpu.sync_copy(zstage, acc.at[pl.ds(s * 8, 8)])
    plsc.subcore_barrier()

    rr = c * 32 + s * 2
    pltpu.sync_copy(sidx.at[pl.ds(rr, 2)], sbuf)
    pltpu.sync_copy(sdstl.at[pl.ds(rr, 2)], dbuf)
    gd = [None] * 2
    sd = [None] * 2
    for k in range(2):
        gd[k] = pltpu.async_copy(h3.at[sbuf.at[k]], rbuf.at[k], gsem)
    for j in range(2):
        gd[j].wait()
        sd[j] = pltpu.async_copy(rbuf.at[j], acc.at[dbuf.at[j]], ssem,
                                 add=True)
    for j in range(2):
        sd[j].wait()
    plsc.subcore_barrier()

    r0 = c * 128 + s * 8
    pltpu.sync_copy(acc.at[pl.ds(s * 8, 8)], zstage)
    pltpu.sync_copy(zstage, out.at[pl.ds(r0, 8)])


_MESH = None


def _mesh():
    global _MESH
    if _MESH is None:
        _MESH = plsc.VectorSubcoreMesh(core_axis_name="c",
                                       subcore_axis_name="s", num_cores=2,
                                       num_subcores=16)
    return _MESH


def _sc_deg(dstp):
    return pl.kernel(
        _deg_body,
        out_type=[jax.ShapeDtypeStruct((ACC_DEG,), jnp.float32),
                  jax.ShapeDtypeStruct((ACC_DEG,), jnp.float32)],
        mesh=_mesh(),
        compiler_params=pltpu.CompilerParams(use_tc_tiling_on_sc=False),
        scratch_types=[
            pltpu.VMEM((2, 512), jnp.int32),
            pltpu.VMEM((512,), jnp.float32),
            pltpu.VMEM((8192,), jnp.float32),
            pltpu.VMEM_SHARED((ACC_DEG,), jnp.float32),
            pltpu.SemaphoreType.DMA,
        ],
    )(dstp)


def _sc_prop1(vp1, srcp, dstp):
    return pl.kernel(
        _prop1_body,
        out_type=jax.ShapeDtypeStruct((2, ACC_ROWS, 16), jnp.float32),
        mesh=_mesh(),
        compiler_params=pltpu.CompilerParams(use_tc_tiling_on_sc=False),
        scratch_types=[
            pltpu.VMEM((2, 512), jnp.int32),
            pltpu.VMEM((2, 512), jnp.int32),
            pltpu.VMEM((NB, 512, 16), jnp.float32),
            pltpu.VMEM_SHARED((ACC_ROWS, 16), jnp.float32),
            pltpu.SemaphoreType.DMA,
            pltpu.SemaphoreType.DMA,
        ],
    )(vp1, srcp, dstp)


def _sc_prop23(vps, srcp, dstp):
    return pl.kernel(
        _prop23_body,
        out_type=jax.ShapeDtypeStruct((4, ACC_ROWS, 16), jnp.float32),
        mesh=_mesh(),
        compiler_params=pltpu.CompilerParams(use_tc_tiling_on_sc=False),
        scratch_types=[
            pltpu.VMEM((4, 512), jnp.int32),
            pltpu.VMEM((4, 512), jnp.int32),
            pltpu.VMEM((NB, 512, 16), jnp.float32),
            pltpu.VMEM_SHARED((ACC_ROWS, 16), jnp.float32),
            pltpu.SemaphoreType.DMA,
            pltpu.SemaphoreType.DMA,
        ],
    )(vps, srcp, dstp)


def _sc_sheet(h3, sidx, sdstl):
    return pl.kernel(
        _sheet_body,
        out_type=jax.ShapeDtypeStruct((NSHEET, HID), jnp.float32),
        mesh=_mesh(),
        compiler_params=pltpu.CompilerParams(use_tc_tiling_on_sc=False),
        scratch_types=[
            pltpu.VMEM((2, 512), jnp.int32),
            pltpu.VMEM((2, 512), jnp.int32),
            pltpu.VMEM((2, 512, HID), jnp.float32),
            pltpu.VMEM((8, HID), jnp.float32),
            pltpu.VMEM_SHARED((128, HID), jnp.float32),
            pltpu.SemaphoreType.DMA,
            pltpu.SemaphoreType.DMA,
        ],
    )(h3, sidx, sdstl)


_BLK = 1000
_GRID = NN // _BLK


def _prep_body(d0, d1, xp, dis, vp1):
    dv = lax.rsqrt(d0[...] + d1[...] + 1.0)
    dis[...] = dv
    vp1[...] = dv * xp[...]


def _tc_prep(d0, d1, xp):
    return pl.pallas_call(
        _prep_body,
        grid=(_GRID,),
        in_specs=[
            pl.BlockSpec((_BLK, 1), lambda i: (i, 0)),
            pl.BlockSpec((_BLK, 1), lambda i: (i, 0)),
            pl.BlockSpec((_BLK, 16), lambda i: (i, 0)),
        ],
        out_specs=[
            pl.BlockSpec((_BLK, 1), lambda i: (i, 0)),
            pl.BlockSpec((_BLK, 16), lambda i: (i, 0)),
        ],
        out_shape=[
            jax.ShapeDtypeStruct((NN, 1), jnp.float32),
            jax.ShapeDtypeStruct((NN, 16), jnp.float32),
        ],
    )(d0, d1, xp)


def _comb1_body(o, vp1, dis, w0, w1, b0, vp2):
    t = dis[...] * (o[0] + o[1] - vp1[...])
    h1 = jnp.maximum(jnp.dot(t, w0[...]) + b0[...], 0.0)
    v = dis[...] * jnp.dot(h1, w1[...])
    for cidx in range(4):
        vp2[cidx] = v[:, cidx * 16:(cidx + 1) * 16]


def _tc_comb1(o, vp1, dis, w0p, w1, b0):
    return pl.pallas_call(
        _comb1_body,
        grid=(_GRID,),
        in_specs=[
            pl.BlockSpec((2, _BLK, 16), lambda i: (0, i, 0)),
            pl.BlockSpec((_BLK, 16), lambda i: (i, 0)),
            pl.BlockSpec((_BLK, 1), lambda i: (i, 0)),
            pl.BlockSpec((16, HID), lambda i: (0, 0)),
            pl.BlockSpec((HID, HID), lambda i: (0, 0)),
            pl.BlockSpec((1, HID), lambda i: (0, 0)),
        ],
        out_specs=pl.BlockSpec((4, _BLK, 16), lambda i: (0, i, 0)),
        out_shape=jax.ShapeDtypeStruct((4, ACC_ROWS, 16), jnp.float32),
    )(o, vp1, dis, w0p, w1, b0)


def _comb2_body(o, dis, w2, b1, vp3):
    x = jnp.concatenate([o[i] for i in range(4)], axis=1)
    h2 = jnp.maximum(dis[...] * x + b1[...], 0.0)
    v = dis[...] * jnp.dot(h2, w2[...])
    for cidx in range(4):
        vp3[cidx] = v[:, cidx * 16:(cidx + 1) * 16]


def _tc_comb2(o, dis, w2, b1):
    return pl.pallas_call(
        _comb2_body,
        grid=(_GRID,),
        in_specs=[
            pl.BlockSpec((4, _BLK, 16), lambda i: (0, i, 0)),
            pl.BlockSpec((_BLK, 1), lambda i: (i, 0)),
            pl.BlockSpec((HID, HID), lambda i: (0, 0)),
            pl.BlockSpec((1, HID), lambda i: (0, 0)),
        ],
        out_specs=pl.BlockSpec((4, _BLK, 16), lambda i: (0, i, 0)),
        out_shape=jax.ShapeDtypeStruct((4, ACC_ROWS, 16), jnp.float32),
    )(o, dis, w2, b1)


def _comb3_body(o, dis, b2, h3, gs):
    x = jnp.concatenate([o[i] for i in range(4)], axis=1)
    h = jnp.maximum(dis[...] * x + b2[...], 0.0)
    h3[...] = h

    @pl.when(pl.program_id(0) == 0)
    def _():
        gs[...] = jnp.zeros_like(gs)
    gs[...] += jnp.sum(h, axis=0, keepdims=True)


def _tc_comb3(o, dis, b2):
    return pl.pallas_call(
        _comb3_body,
        grid=(_GRID,),
        in_specs=[
            pl.BlockSpec((4, _BLK, 16), lambda i: (0, i, 0)),
            pl.BlockSpec((_BLK, 1), lambda i: (i, 0)),
            pl.BlockSpec((1, HID), lambda i: (0, 0)),
        ],
        out_specs=[
            pl.BlockSpec((_BLK, HID), lambda i: (i, 0)),
            pl.BlockSpec((1, HID), lambda i: (0, 0)),
        ],
        out_shape=[
            jax.ShapeDtypeStruct((NN, HID), jnp.float32),
            jax.ShapeDtypeStruct((1, HID), jnp.float32),
        ],
    )(o, dis, b2)


def _mlp_body(ssum, gs, sfp, wg1, bg1, wg2, bg2, wfa, wfb, bf,
              wq1a, wq1b, bq1, wq2, bq2, out):
    se = ssum[...] * (1.0 / LSHEET)
    ge = gs[...] * (1.0 / NN)
    geo = jnp.dot(jnp.maximum(jnp.dot(sfp[...], wg1[...]) + bg1[...], 0.0),
                  wg2[...]) + bg2[...]
    fused = jnp.maximum(jnp.dot(se, wfa[...]) + jnp.dot(geo, wfb[...])
                        + bf[...], 0.0)
    gq = jnp.dot(ge, wq1b[...])
    z = jnp.maximum(jnp.dot(fused, wq1a[...]) + gq + bq1[...], 0.0)
    out[...] = jnp.dot(z, wq2[...]) + bq2[...]


def _tc_mlp(*args):
    return pl.pallas_call(
        _mlp_body,
        out_shape=jax.ShapeDtypeStruct((NSHEET, 1), jnp.float32),
    )(*args)


def kernel(x, edge_index, batch, sheet_node_idx, sheet_features,
           W0, b0, W1, b1, W2, b2, Wg1, bg1, Wg2, bg2, Wf, bf,
           Wq1, bq1, Wq2, bq2):
    i32 = jnp.int32
    src = edge_index[0]
    dst = edge_index[1]
    npad = EROWS * 512 - EE
    srcp = jnp.concatenate([src, jnp.zeros((npad,), i32)]).reshape(EROWS, 512)
    dstp = jnp.concatenate([dst, jnp.full((npad,), PAD_ROW, i32)]
                           ).reshape(EROWS, 512)
    xp = jnp.pad(x, ((0, 0), (0, 6)))
    w0p = jnp.pad(W0, ((0, 6), (0, 0)))
    wg1p = jnp.pad(Wg1, ((0, 6), (0, 0)))
    sfp = jnp.pad(sheet_features, ((0, 0), (0, 6)))
    sdstl = jnp.broadcast_to(
        (jnp.arange(NSHEET, dtype=i32) % 128)[:, None],
        (NSHEET, LSHEET)).reshape(64, 512)
    sidxr = sheet_node_idx.reshape(64, 512)
    b0r = b0.reshape(1, HID)
    b1r = b1.reshape(1, HID)
    b2r = b2.reshape(1, HID)
    bg1r = bg1.reshape(1, HID)
    bg2r = bg2.reshape(1, HID)
    bfr = bf.reshape(1, HID)
    bq1r = bq1.reshape(1, HID)
    bq2r = bq2.reshape(1, 1)
    wfa, wfb = Wf[:HID], Wf[HID:]
    wq1a, wq1b = Wq1[:HID], Wq1[HID:]

    dg0, dg1 = _sc_deg(dstp)
    d0 = dg0[:NN].reshape(NN, 1)
    d1 = dg1[:NN].reshape(NN, 1)
    dis, vp1 = _tc_prep(d0, d1, xp)

    o1 = _sc_prop1(vp1, srcp, dstp)
    vp2 = _tc_comb1(o1, vp1, dis, w0p, W1, b0r)

    o2 = _sc_prop23(vp2, srcp, dstp)
    vp3 = _tc_comb2(o2, dis, W2, b1r)

    o3 = _sc_prop23(vp3, srcp, dstp)
    h3, gs = _tc_comb3(o3, dis, b2r)

    ssum = _sc_sheet(h3, sidxr, sdstl)
    q = _tc_mlp(ssum, gs, sfp, wg1p, bg1r, Wg2, bg2r, wfa, wfb, bfr,
                wq1a, wq1b, bq1r, Wq2, bq2r)
    return jnp.squeeze(q, -1)
